# baseline (speedup 1.0000x reference)
import functools

import numpy as np
import jax
import jax.numpy as jnp
from jax import lax
from jax.experimental import pallas as pl
from jax.experimental.pallas import tpu as pltpu
from jax.experimental.pallas import tpu_sc as plsc

D = 128
NC = 2
NS = 16
CH = 128


_z = np.int32(0)


def _rup(x, m):
    return (x + m - 1) // m * m


@functools.cache
def _sc_segsum(K, R, n_src, ch, nbuf):
    mesh = plsc.VectorSubcoreMesh(core_axis_name="c", subcore_axis_name="s")
    rs = R // NS

    @functools.partial(
        pl.kernel,
        mesh=mesh,
        out_type=jax.ShapeDtypeStruct((NC, R, D), jnp.float32),
        scratch_types=[
            pltpu.VMEM((K, ch), jnp.int32),
            pltpu.VMEM((K, ch), jnp.int32),
            [pltpu.VMEM((ch, D), jnp.float32)] * nbuf,
            pltpu.VMEM_SHARED((R, D), jnp.float32),
            [pltpu.SemaphoreType.DMA] * nbuf,
        ],
    )
    def k(m_hbm, src_hbm, dst_hbm, z_hbm, out_hbm, src_v, dst_v, rows_v, acc, sems):
        cid = lax.axis_index("c")
        sid = lax.axis_index("s")
        tid = cid * NS + sid
        pltpu.sync_copy(z_hbm, acc.at[pl.ds(sid * rs, rs)])
        pltpu.sync_copy(src_hbm.at[tid], src_v)
        pltpu.sync_copy(dst_hbm.at[tid], dst_v)
        plsc.subcore_barrier()

        for j in range(nbuf):
            pltpu.async_copy(m_hbm.at[src_v.at[np.int32(j)]], rows_v[j], sems[j])

        def body(kb, carry):
            for j in range(nbuf):
                i = kb * np.int32(nbuf) + np.int32(j)
                pltpu.make_async_copy(m_hbm.at[src_v.at[i]],
                                      rows_v[j], sems[j]).wait()
                pltpu.sync_copy(rows_v[j], acc.at[dst_v.at[i]], add=True)
                pltpu.async_copy(m_hbm.at[src_v.at[i + np.int32(nbuf)]],
                                 rows_v[j], sems[j])
            return carry

        if K > nbuf:
            lax.fori_loop(jnp.int32(0), jnp.int32(K // nbuf - 1), body,
                          jnp.int32(0))
        for j in range(nbuf):
            i = np.int32(K - nbuf + j)
            pltpu.make_async_copy(m_hbm.at[src_v.at[i]],
                                  rows_v[j], sems[j]).wait()
            pltpu.sync_copy(rows_v[j], acc.at[dst_v.at[i]], add=True)
        plsc.subcore_barrier()
        pltpu.sync_copy(acc.at[pl.ds(sid * rs, rs)],
                        out_hbm.at[cid, pl.ds(sid * rs, rs)])

    return k


KB = 16


@functools.cache
def _sc_segsum_big(K, R, n_src):
    mesh = plsc.VectorSubcoreMesh(core_axis_name="c", subcore_axis_name="s")
    rs = R // NS
    NB = K // KB

    @functools.partial(
        pl.kernel,
        mesh=mesh,
        out_type=jax.ShapeDtypeStruct((NC, R, D), jnp.float32),
        scratch_types=[
            pltpu.VMEM((KB, CH), jnp.int32),
            pltpu.VMEM((KB, CH), jnp.int32),
            [pltpu.VMEM((CH, D), jnp.float32)] * 2,
            pltpu.VMEM_SHARED((R, D), jnp.float32),
            [pltpu.SemaphoreType.DMA] * 2,
        ],
    )
    def k(m_hbm, src_hbm, dst_hbm, z_hbm, out_hbm, src_v, dst_v, rows_v, acc, sems):
        cid = lax.axis_index("c")
        sid = lax.axis_index("s")
        tid = cid * NS + sid
        pltpu.sync_copy(z_hbm, acc.at[pl.ds(sid * rs, rs)])
        plsc.subcore_barrier()

        def block(b, carry):
            pltpu.sync_copy(src_hbm.at[tid, pl.ds(b * np.int32(KB), KB)], src_v)
            pltpu.sync_copy(dst_hbm.at[tid, pl.ds(b * np.int32(KB), KB)], dst_v)
            for j in range(2):
                pltpu.async_copy(m_hbm.at[src_v.at[np.int32(j)]], rows_v[j],
                                 sems[j])

            def body(q, carry2):
                for j in range(2):
                    i = q * np.int32(2) + np.int32(j)
                    pltpu.make_async_copy(m_hbm.at[src_v.at[i]],
                                          rows_v[j], sems[j]).wait()
                    pltpu.sync_copy(rows_v[j], acc.at[dst_v.at[i]], add=True)
                    pltpu.async_copy(m_hbm.at[src_v.at[i + np.int32(2)]],
                                     rows_v[j], sems[j])
                return carry2

            lax.fori_loop(jnp.int32(0), jnp.int32(KB // 2 - 1), body,
                          jnp.int32(0))
            for j in range(2):
                i = np.int32(KB - 2 + j)
                pltpu.make_async_copy(m_hbm.at[src_v.at[i]],
                                      rows_v[j], sems[j]).wait()
                pltpu.sync_copy(rows_v[j], acc.at[dst_v.at[i]], add=True)
            return carry

        lax.fori_loop(jnp.int32(0), jnp.int32(NB), block, jnp.int32(0))
        plsc.subcore_barrier()
        pltpu.sync_copy(acc.at[pl.ds(sid * rs, rs)],
                        out_hbm.at[cid, pl.ds(sid * rs, rs)])

    return k


def _segsum(m, src, dst, n_out):
    e = src.shape[0]
    n_src = m.shape[0]
    R = _rup(n_out + 1, 8 * NS)
    big = R * D >= 2 ** 20
    mult = NC * NS * CH * (KB if big else 4)
    ep = _rup(e, mult)
    K = ep // (NC * NS * CH)
    pad = ep - e
    src_p = jnp.concatenate([src, jnp.zeros((pad,), jnp.int32)]).reshape(
        NC * NS, K, CH)
    dst_p = jnp.concatenate([dst, jnp.full((pad,), n_out, jnp.int32)]).reshape(
        NC * NS, K, CH)
    zrows = jnp.zeros((R // NS, D), jnp.float32)
    if big:
        return _sc_segsum_big(K, R, n_src)(m, src_p, dst_p, zrows)
    return _sc_segsum(K, R, n_src, CH, 4)(m, src_p, dst_p, zrows)


@functools.cache
def _sc_rowgather(K, n_src):
    mesh = plsc.VectorSubcoreMesh(core_axis_name="c", subcore_axis_name="s")

    @functools.partial(
        pl.kernel,
        mesh=mesh,
        out_type=jax.ShapeDtypeStruct((NC * NS * K * CH, D), jnp.float32),
        scratch_types=[
            pltpu.VMEM((K, CH), jnp.int32),
            [pltpu.VMEM((CH, D), jnp.float32)] * K,
            [pltpu.SemaphoreType.DMA] * K,
        ],
    )
    def k(u_hbm, widx_hbm, out_hbm, widx_v, rows_v, sems):
        cid = lax.axis_index("c")
        sid = lax.axis_index("s")
        tid = cid * NS + sid
        pltpu.sync_copy(widx_hbm.at[tid], widx_v)
        for j in range(K):
            pltpu.async_copy(u_hbm.at[widx_v.at[np.int32(j)]], rows_v[j],
                             sems[j])
        for j in range(K):
            pltpu.make_async_copy(u_hbm.at[widx_v.at[np.int32(j)]],
                                  rows_v[j], sems[j]).wait()
            pltpu.sync_copy(rows_v[j],
                            out_hbm.at[pl.ds((tid * K + j) * CH, CH)])

    return k


def _rowgather(table, widx):
    n = widx.shape[0]
    npad = _rup(n, NC * NS * CH)
    K = npad // (NC * NS * CH)
    widx_p = jnp.concatenate(
        [widx, jnp.zeros((npad - n,), jnp.int32)]).reshape(NC * NS, K, CH)
    return _sc_rowgather(K, table.shape[0])(table, widx_p)


def _dot(a, b):
    return lax.dot_general(a, b, (((1,), (0,)), ((), ())),
                           precision=lax.Precision.HIGHEST,
                           preferred_element_type=jnp.float32)


def _ln(z):
    mu = jnp.mean(z, axis=-1, keepdims=True)
    var = jnp.mean((z - mu) ** 2, axis=-1, keepdims=True)
    return (z - mu) * lax.rsqrt(var + 1e-5)


def _ln_mm2_body(z_ref, wm_ref, ws_ref, m_ref, s_ref):
    h = _ln(z_ref[...])
    m_ref[...] = _dot(h, wm_ref[...])
    s_ref[...] = _dot(h, ws_ref[...])


@functools.cache
def _ln_mm2(n, bn):
    grid = n // bn
    w_spec = pl.BlockSpec((D, D), lambda i: (_z, _z))
    r_spec = pl.BlockSpec((bn, D), lambda i: (i, _z))
    return pl.pallas_call(
        _ln_mm2_body,
        grid=(grid,),
        in_specs=[r_spec, w_spec, w_spec],
        out_specs=[r_spec, r_spec],
        out_shape=[jax.ShapeDtypeStruct((n, D), jnp.float32)] * 2,
    )


def _combine1_body(s_ref, aggp_ref, wup_ref, h_ref, u_ref, *, n):
    hc = s_ref[...] + aggp_ref[0, :n, :] + aggp_ref[1, :n, :]
    h_ref[...] = hc
    u_ref[...] = _dot(hc, wup_ref[...])


@functools.cache
def _combine1(n, R):
    spec = pl.BlockSpec((n, D), lambda: (_z, _z))
    return pl.pallas_call(
        functools.partial(_combine1_body, n=n),
        in_specs=[spec,
                  pl.BlockSpec((NC, R, D), lambda: (_z, _z, _z)),
                  pl.BlockSpec((D, D), lambda: (_z, _z))],
        out_specs=[spec, spec],
        out_shape=[jax.ShapeDtypeStruct((n, D), jnp.float32)] * 2,
    )


def _assemble0_body(s_ref, aggp_ref, inp_ref, msk_ref, wup_ref, bup_ref, o_ref,
                    *, final_ln):
    hc = s_ref[...] + aggp_ref[0] + aggp_ref[1]
    z = (hc + msk_ref[...] * inp_ref[...] + _dot(hc, wup_ref[...])
         + bup_ref[...])
    o_ref[...] = _ln(z) if final_ln else z


@functools.cache
def _assemble0(n, bn, R, final_ln):
    grid = n // bn
    r_spec = pl.BlockSpec((bn, D), lambda i: (i, _z))
    p_spec = pl.BlockSpec((NC, bn, D), lambda i: (_z, i, _z))
    return pl.pallas_call(
        functools.partial(_assemble0_body, final_ln=final_ln),
        grid=(grid,),
        in_specs=[r_spec, p_spec, r_spec,
                  pl.BlockSpec((bn, 1), lambda i: (i, _z)),
                  pl.BlockSpec((D, D), lambda i: (_z, _z)),
                  pl.BlockSpec((1, D), lambda i: (_z, _z))],
        out_specs=r_spec,
        out_shape=jax.ShapeDtypeStruct((n, D), jnp.float32),
    )


def _ln_only_body(z_ref, o_ref):
    o_ref[...] = _ln(z_ref[...])


@functools.cache
def _ln_only(n):
    spec = pl.BlockSpec((n, D), lambda: (_z, _z))
    return pl.pallas_call(
        _ln_only_body,
        in_specs=[spec],
        out_specs=spec,
        out_shape=jax.ShapeDtypeStruct((n, D), jnp.float32),
    )


def kernel(hn0, hn1, Wself, Wmsg, Wup, bup, edge_index0, edge_index1, idx1):
    n0, _ = hn0.shape
    n1, _ = hn1.shape
    L = Wself.shape[0]
    out_dt = jnp.result_type(hn0.dtype, Wself.dtype, Wup.dtype)
    src0 = edge_index0[0].astype(jnp.int32)
    dst0 = edge_index0[1].astype(jnp.int32)
    src1 = edge_index1[0].astype(jnp.int32)
    dst1 = edge_index1[1].astype(jnp.int32)
    idx1 = idx1.astype(jnp.int32)
    Wself = Wself.astype(jnp.float32)
    Wmsg = Wmsg.astype(jnp.float32)
    Wup = Wup.astype(jnp.float32)
    bup = bup.astype(jnp.float32)

    ar = jnp.arange(n1, dtype=jnp.int32)
    winner = jnp.full((n0,), -1, jnp.int32).at[idx1].max(ar,
                                                         mode='promise_in_bounds')
    mask0 = (winner >= 0).astype(jnp.float32)[:, None]
    widx = jnp.maximum(winner, 0)

    bn0 = 1000
    R0 = _rup(n0 + 1, 8 * NS)
    R1 = _rup(n1 + 1, 8 * NS)

    z0, z1 = hn0.astype(jnp.float32), hn1.astype(jnp.float32)
    for l in range(L):
        m0, s0 = _ln_mm2(n0, bn0)(z0, Wmsg[l, 0], Wself[l, 0])
        m1, s1 = _ln_mm2(n1, n1)(z1, Wmsg[l, 1], Wself[l, 1])
        agg0p = _segsum(m0, src0, dst0, n0)
        agg1p = _segsum(m1, src1, dst1, n1)
        h1c, u1 = _combine1(n1, R1)(s1, agg1p, Wup[l, :D])
        inp = _rowgather(u1, widx)
        z0 = _assemble0(n0, bn0, R0, l == L - 1)(
            s0, agg0p, inp, mask0, Wup[l, D:], bup[l][None, :])
        z1 = h1c
    return (z0.astype(out_dt), _ln_only(n1)(z1).astype(out_dt))


# speedup vs baseline: 1.1265x; 1.1265x over previous
import functools

import numpy as np
import jax
import jax.numpy as jnp
from jax import lax
from jax.experimental import pallas as pl
from jax.experimental.pallas import tpu as pltpu
from jax.experimental.pallas import tpu_sc as plsc

D = 128
NC = 2
NS = 16
CH = 128


_z = np.int32(0)


def _rup(x, m):
    return (x + m - 1) // m * m


@functools.cache
def _sc_segsum(K, R, n_src):
    mesh = plsc.VectorSubcoreMesh(core_axis_name="c", subcore_axis_name="s")
    rs = R // NS

    @functools.partial(
        pl.kernel,
        mesh=mesh,
        out_type=jax.ShapeDtypeStruct((NC, R, D), jnp.float32),
        scratch_types=[
            pltpu.VMEM((K, CH), jnp.int32),
            pltpu.VMEM((K, CH), jnp.int32),
            pltpu.VMEM((CH, D), jnp.float32),
            pltpu.VMEM_SHARED((R, D), jnp.float32),
            pltpu.SemaphoreType.DMA,
        ],
    )
    def k(m_hbm, src_hbm, dst_hbm, z_hbm, out_hbm, src_v, dst_v, rows_v, acc, sem):
        cid = lax.axis_index("c")
        sid = lax.axis_index("s")
        tid = cid * NS + sid
        pltpu.sync_copy(z_hbm, acc.at[pl.ds(sid * rs, rs)])
        pltpu.sync_copy(src_hbm.at[tid], src_v)
        pltpu.sync_copy(dst_hbm.at[tid], dst_v)
        plsc.subcore_barrier()

        def body(i, carry):
            pltpu.async_copy(m_hbm.at[src_v.at[i]], rows_v, sem).wait()
            pltpu.sync_copy(rows_v, acc.at[dst_v.at[i]], add=True)
            return carry

        lax.fori_loop(jnp.int32(0), jnp.int32(K), body, jnp.int32(0))
        plsc.subcore_barrier()
        pltpu.sync_copy(acc.at[pl.ds(sid * rs, rs)],
                        out_hbm.at[cid, pl.ds(sid * rs, rs)])

    return k


def _segsum(m, src, dst, n_out):
    e = src.shape[0]
    n_src = m.shape[0]
    R = _rup(n_out + 1, 8 * NS)
    ep = _rup(e, NC * NS * CH)
    K = ep // (NC * NS * CH)
    pad = ep - e
    src_p = jnp.concatenate([src, jnp.zeros((pad,), jnp.int32)]).reshape(
        NC * NS, K, CH)
    dst_p = jnp.concatenate([dst, jnp.full((pad,), n_out, jnp.int32)]).reshape(
        NC * NS, K, CH)
    zrows = jnp.zeros((R // NS, D), jnp.float32)
    return _sc_segsum(K, R, n_src)(m, src_p, dst_p, zrows)


@functools.cache
def _sc_rowgather(K, n_src):
    mesh = plsc.VectorSubcoreMesh(core_axis_name="c", subcore_axis_name="s")

    @functools.partial(
        pl.kernel,
        mesh=mesh,
        out_type=jax.ShapeDtypeStruct((NC * NS * K * CH, D), jnp.float32),
        scratch_types=[
            pltpu.VMEM((K, CH), jnp.int32),
            [pltpu.VMEM((CH, D), jnp.float32)] * K,
            [pltpu.SemaphoreType.DMA] * K,
        ],
    )
    def k(u_hbm, widx_hbm, out_hbm, widx_v, rows_v, sems):
        cid = lax.axis_index("c")
        sid = lax.axis_index("s")
        tid = cid * NS + sid
        pltpu.sync_copy(widx_hbm.at[tid], widx_v)
        for j in range(K):
            pltpu.async_copy(u_hbm.at[widx_v.at[np.int32(j)]], rows_v[j],
                             sems[j])
        for j in range(K):
            pltpu.make_async_copy(u_hbm.at[widx_v.at[np.int32(j)]],
                                  rows_v[j], sems[j]).wait()
            pltpu.sync_copy(rows_v[j],
                            out_hbm.at[pl.ds((tid * K + j) * CH, CH)])

    return k


def _rowgather(table, widx):
    n = widx.shape[0]
    npad = _rup(n, NC * NS * CH)
    K = npad // (NC * NS * CH)
    widx_p = jnp.concatenate(
        [widx, jnp.zeros((npad - n,), jnp.int32)]).reshape(NC * NS, K, CH)
    return _sc_rowgather(K, table.shape[0])(table, widx_p)


def _dot(a, b):
    return lax.dot_general(a, b, (((1,), (0,)), ((), ())),
                           precision=lax.Precision.HIGHEST,
                           preferred_element_type=jnp.float32)


def _ln(z):
    mu = jnp.mean(z, axis=-1, keepdims=True)
    var = jnp.mean((z - mu) ** 2, axis=-1, keepdims=True)
    return (z - mu) * lax.rsqrt(var + 1e-5)


def _ln_mm2_body(z_ref, wm_ref, ws_ref, m_ref, s_ref):
    h = _ln(z_ref[...])
    m_ref[...] = _dot(h, wm_ref[...])
    s_ref[...] = _dot(h, ws_ref[...])


@functools.cache
def _ln_mm2(n, bn):
    grid = n // bn
    w_spec = pl.BlockSpec((D, D), lambda i: (_z, _z))
    r_spec = pl.BlockSpec((bn, D), lambda i: (i, _z))
    return pl.pallas_call(
        _ln_mm2_body,
        grid=(grid,),
        in_specs=[r_spec, w_spec, w_spec],
        out_specs=[r_spec, r_spec],
        out_shape=[jax.ShapeDtypeStruct((n, D), jnp.float32)] * 2,
    )


def _combine1_body(s_ref, aggp_ref, wup_ref, h_ref, u_ref, *, n):
    hc = s_ref[...] + aggp_ref[0, :n, :] + aggp_ref[1, :n, :]
    h_ref[...] = hc
    u_ref[...] = _dot(hc, wup_ref[...])


@functools.cache
def _combine1(n, R):
    spec = pl.BlockSpec((n, D), lambda: (_z, _z))
    return pl.pallas_call(
        functools.partial(_combine1_body, n=n),
        in_specs=[spec,
                  pl.BlockSpec((NC, R, D), lambda: (_z, _z, _z)),
                  pl.BlockSpec((D, D), lambda: (_z, _z))],
        out_specs=[spec, spec],
        out_shape=[jax.ShapeDtypeStruct((n, D), jnp.float32)] * 2,
    )


def _assemble0_body(s_ref, aggp_ref, inp_ref, msk_ref, wup_ref, bup_ref, o_ref,
                    *, final_ln):
    hc = s_ref[...] + aggp_ref[0] + aggp_ref[1]
    z = (hc + msk_ref[...] * inp_ref[...] + _dot(hc, wup_ref[...])
         + bup_ref[...])
    o_ref[...] = _ln(z) if final_ln else z


@functools.cache
def _assemble0(n, bn, R, final_ln):
    grid = n // bn
    r_spec = pl.BlockSpec((bn, D), lambda i: (i, _z))
    p_spec = pl.BlockSpec((NC, bn, D), lambda i: (_z, i, _z))
    return pl.pallas_call(
        functools.partial(_assemble0_body, final_ln=final_ln),
        grid=(grid,),
        in_specs=[r_spec, p_spec, r_spec,
                  pl.BlockSpec((bn, 1), lambda i: (i, _z)),
                  pl.BlockSpec((D, D), lambda i: (_z, _z)),
                  pl.BlockSpec((1, D), lambda i: (_z, _z))],
        out_specs=r_spec,
        out_shape=jax.ShapeDtypeStruct((n, D), jnp.float32),
    )


def _ln_only_body(z_ref, o_ref):
    o_ref[...] = _ln(z_ref[...])


@functools.cache
def _ln_only(n):
    spec = pl.BlockSpec((n, D), lambda: (_z, _z))
    return pl.pallas_call(
        _ln_only_body,
        in_specs=[spec],
        out_specs=spec,
        out_shape=jax.ShapeDtypeStruct((n, D), jnp.float32),
    )


def kernel(hn0, hn1, Wself, Wmsg, Wup, bup, edge_index0, edge_index1, idx1):
    n0, _ = hn0.shape
    n1, _ = hn1.shape
    L = Wself.shape[0]
    out_dt = jnp.result_type(hn0.dtype, Wself.dtype, Wup.dtype)
    src0 = edge_index0[0].astype(jnp.int32)
    dst0 = edge_index0[1].astype(jnp.int32)
    src1 = edge_index1[0].astype(jnp.int32)
    dst1 = edge_index1[1].astype(jnp.int32)
    idx1 = idx1.astype(jnp.int32)
    Wself = Wself.astype(jnp.float32)
    Wmsg = Wmsg.astype(jnp.float32)
    Wup = Wup.astype(jnp.float32)
    bup = bup.astype(jnp.float32)

    ar = jnp.arange(n1, dtype=jnp.int32)
    winner = jnp.full((n0,), -1, jnp.int32).at[idx1].max(ar,
                                                         mode='promise_in_bounds')
    mask0 = (winner >= 0).astype(jnp.float32)[:, None]
    widx = jnp.maximum(winner, 0)

    bn0 = 1000
    R0 = _rup(n0 + 1, 8 * NS)
    R1 = _rup(n1 + 1, 8 * NS)

    z0, z1 = hn0.astype(jnp.float32), hn1.astype(jnp.float32)
    for l in range(L):
        m0, s0 = _ln_mm2(n0, bn0)(z0, Wmsg[l, 0], Wself[l, 0])
        m1, s1 = _ln_mm2(n1, n1)(z1, Wmsg[l, 1], Wself[l, 1])
        agg0p = _segsum(m0, src0, dst0, n0)
        agg1p = _segsum(m1, src1, dst1, n1)
        h1c, u1 = _combine1(n1, R1)(s1, agg1p, Wup[l, :D])
        inp = _rowgather(u1, widx)
        z0 = _assemble0(n0, bn0, R0, l == L - 1)(
            s0, agg0p, inp, mask0, Wup[l, D:], bup[l][None, :])
        z1 = h1c
    return (z0.astype(out_dt), _ln_only(n1)(z1).astype(out_dt))


# speedup vs baseline: 1.7253x; 1.5315x over previous
import functools

import numpy as np
import jax
import jax.numpy as jnp
from jax import lax
from jax.experimental import pallas as pl
from jax.experimental.pallas import tpu as pltpu
from jax.experimental.pallas import tpu_sc as plsc

D = 128
NC = 2
NS = 16
CH = 128


_z = np.int32(0)


def _rup(x, m):
    return (x + m - 1) // m * m


@functools.cache
def _sc_segsum(K, R, n_src):
    mesh = plsc.VectorSubcoreMesh(core_axis_name="c", subcore_axis_name="s")
    rs = R // NS

    @functools.partial(
        pl.kernel,
        mesh=mesh,
        out_type=jax.ShapeDtypeStruct((NC, R, D), jnp.float32),
        scratch_types=[
            pltpu.VMEM((K, CH), jnp.int32),
            pltpu.VMEM((K, CH), jnp.int32),
            pltpu.VMEM((CH, D), jnp.float32),
            pltpu.VMEM_SHARED((R, D), jnp.float32),
            pltpu.SemaphoreType.DMA,
        ],
    )
    def k(m_hbm, src_hbm, dst_hbm, z_hbm, out_hbm, src_v, dst_v, rows_v, acc, sem):
        cid = lax.axis_index("c")
        sid = lax.axis_index("s")
        tid = cid * NS + sid
        pltpu.sync_copy(z_hbm, acc.at[pl.ds(sid * rs, rs)])
        pltpu.sync_copy(src_hbm.at[tid], src_v)
        pltpu.sync_copy(dst_hbm.at[tid], dst_v)
        plsc.subcore_barrier()

        def body(i, carry):
            pltpu.async_copy(m_hbm.at[src_v.at[i]], rows_v, sem).wait()
            pltpu.sync_copy(rows_v, acc.at[dst_v.at[i]], add=True)
            return carry

        lax.fori_loop(jnp.int32(0), jnp.int32(K), body, jnp.int32(0))
        plsc.subcore_barrier()
        pltpu.sync_copy(acc.at[pl.ds(sid * rs, rs)],
                        out_hbm.at[cid, pl.ds(sid * rs, rs)])

    return k


def _segsum(m, src, dst, n_out):
    e = src.shape[0]
    n_src = m.shape[0]
    R = _rup(n_out + 1, 8 * NS)
    ep = _rup(e, NC * NS * CH)
    K = ep // (NC * NS * CH)
    pad = ep - e
    src_p = jnp.concatenate([src, jnp.zeros((pad,), jnp.int32)]).reshape(
        NC * NS, K, CH)
    dst_p = jnp.concatenate([dst, jnp.full((pad,), n_out, jnp.int32)]).reshape(
        NC * NS, K, CH)
    zrows = jnp.zeros((R // NS, D), jnp.float32)
    return _sc_segsum(K, R, n_src)(m, src_p, dst_p, zrows)


@functools.cache
def _sc_rowgather(K, n_src):
    mesh = plsc.VectorSubcoreMesh(core_axis_name="c", subcore_axis_name="s")

    @functools.partial(
        pl.kernel,
        mesh=mesh,
        out_type=jax.ShapeDtypeStruct((NC * NS * K * CH, D), jnp.float32),
        scratch_types=[
            pltpu.VMEM((K, CH), jnp.int32),
            [pltpu.VMEM((CH, D), jnp.float32)] * K,
            [pltpu.SemaphoreType.DMA] * K,
        ],
    )
    def k(u_hbm, widx_hbm, out_hbm, widx_v, rows_v, sems):
        cid = lax.axis_index("c")
        sid = lax.axis_index("s")
        tid = cid * NS + sid
        pltpu.sync_copy(widx_hbm.at[tid], widx_v)
        for j in range(K):
            pltpu.async_copy(u_hbm.at[widx_v.at[np.int32(j)]], rows_v[j],
                             sems[j])
        for j in range(K):
            pltpu.make_async_copy(u_hbm.at[widx_v.at[np.int32(j)]],
                                  rows_v[j], sems[j]).wait()
            pltpu.sync_copy(rows_v[j],
                            out_hbm.at[pl.ds((tid * K + j) * CH, CH)])

    return k


def _rowgather(table, widx):
    n = widx.shape[0]
    npad = _rup(n, NC * NS * CH)
    K = npad // (NC * NS * CH)
    widx_p = jnp.concatenate(
        [widx, jnp.zeros((npad - n,), jnp.int32)]).reshape(NC * NS, K, CH)
    return _sc_rowgather(K, table.shape[0])(table, widx_p)


def _dot(a, b):
    return lax.dot_general(a, b, (((1,), (0,)), ((), ())),
                           precision=lax.Precision.HIGHEST,
                           preferred_element_type=jnp.float32)


def _ln(z):
    mu = jnp.mean(z, axis=-1, keepdims=True)
    var = jnp.mean((z - mu) ** 2, axis=-1, keepdims=True)
    return (z - mu) * lax.rsqrt(var + 1e-5)


def _ln_mm2_body(z_ref, wm_ref, ws_ref, m_ref, s_ref):
    h = _ln(z_ref[...])
    m_ref[...] = _dot(h, wm_ref[...])
    s_ref[...] = _dot(h, ws_ref[...])


@functools.cache
def _ln_mm2(n, bn):
    grid = n // bn
    w_spec = pl.BlockSpec((D, D), lambda i: (_z, _z))
    r_spec = pl.BlockSpec((bn, D), lambda i: (i, _z))
    return pl.pallas_call(
        _ln_mm2_body,
        grid=(grid,),
        in_specs=[r_spec, w_spec, w_spec],
        out_specs=[r_spec, r_spec],
        out_shape=[jax.ShapeDtypeStruct((n, D), jnp.float32)] * 2,
    )


def _combine1_body(s_ref, aggp_ref, wup_ref, keep_ref, h_ref, u_ref, *, n):
    hc = s_ref[...] + aggp_ref[0, :n, :] + aggp_ref[1, :n, :]
    h_ref[...] = hc
    u_ref[...] = _dot(keep_ref[...] * hc, wup_ref[...])


@functools.cache
def _combine1(n, R):
    spec = pl.BlockSpec((n, D), lambda: (_z, _z))
    return pl.pallas_call(
        functools.partial(_combine1_body, n=n),
        in_specs=[spec,
                  pl.BlockSpec((NC, R, D), lambda: (_z, _z, _z)),
                  pl.BlockSpec((D, D), lambda: (_z, _z)),
                  pl.BlockSpec((n, 1), lambda: (_z, _z))],
        out_specs=[spec, spec],
        out_shape=[jax.ShapeDtypeStruct((n, D), jnp.float32)] * 2,
    )


def _assemble0_body(s_ref, aggp_ref, inpp_ref, wup_ref, bup_ref, o_ref,
                    *, final_ln):
    hc = s_ref[...] + aggp_ref[0] + aggp_ref[1]
    z = hc + inpp_ref[0] + inpp_ref[1] + _dot(hc, wup_ref[...]) + bup_ref[...]
    o_ref[...] = _ln(z) if final_ln else z


@functools.cache
def _assemble0(n, bn, R, final_ln):
    grid = n // bn
    r_spec = pl.BlockSpec((bn, D), lambda i: (i, _z))
    p_spec = pl.BlockSpec((NC, bn, D), lambda i: (_z, i, _z))
    return pl.pallas_call(
        functools.partial(_assemble0_body, final_ln=final_ln),
        grid=(grid,),
        in_specs=[r_spec, p_spec, p_spec,
                  pl.BlockSpec((D, D), lambda i: (_z, _z)),
                  pl.BlockSpec((1, D), lambda i: (_z, _z))],
        out_specs=r_spec,
        out_shape=jax.ShapeDtypeStruct((n, D), jnp.float32),
    )


def _ln_only_body(z_ref, o_ref):
    o_ref[...] = _ln(z_ref[...])


@functools.cache
def _ln_only(n):
    spec = pl.BlockSpec((n, D), lambda: (_z, _z))
    return pl.pallas_call(
        _ln_only_body,
        in_specs=[spec],
        out_specs=spec,
        out_shape=jax.ShapeDtypeStruct((n, D), jnp.float32),
    )


def kernel(hn0, hn1, Wself, Wmsg, Wup, bup, edge_index0, edge_index1, idx1):
    n0, _ = hn0.shape
    n1, _ = hn1.shape
    L = Wself.shape[0]
    out_dt = jnp.result_type(hn0.dtype, Wself.dtype, Wup.dtype)
    src0 = edge_index0[0].astype(jnp.int32)
    dst0 = edge_index0[1].astype(jnp.int32)
    src1 = edge_index1[0].astype(jnp.int32)
    dst1 = edge_index1[1].astype(jnp.int32)
    idx1 = idx1.astype(jnp.int32)
    Wself = Wself.astype(jnp.float32)
    Wmsg = Wmsg.astype(jnp.float32)
    Wup = Wup.astype(jnp.float32)
    bup = bup.astype(jnp.float32)

    ar = jnp.arange(n1, dtype=jnp.int32)
    winner = jnp.full((n0,), -1, jnp.int32).at[idx1].max(ar,
                                                         mode='promise_in_bounds')
    keep = (winner[idx1] == ar).astype(jnp.float32)[:, None]

    bn0 = 1000
    R0 = _rup(n0 + 1, 8 * NS)
    R1 = _rup(n1 + 1, 8 * NS)

    z0, z1 = hn0.astype(jnp.float32), hn1.astype(jnp.float32)
    for l in range(L):
        m0, s0 = _ln_mm2(n0, bn0)(z0, Wmsg[l, 0], Wself[l, 0])
        m1, s1 = _ln_mm2(n1, n1)(z1, Wmsg[l, 1], Wself[l, 1])
        agg0p = _segsum(m0, src0, dst0, n0)
        agg1p = _segsum(m1, src1, dst1, n1)
        h1c, u1 = _combine1(n1, R1)(s1, agg1p, Wup[l, :D], keep)
        inpp = _segsum(u1, ar, idx1, n0)
        z0 = _assemble0(n0, bn0, R0, l == L - 1)(
            s0, agg0p, inpp, Wup[l, D:], bup[l][None, :])
        z1 = h1c
    return (z0.astype(out_dt), _ln_only(n1)(z1).astype(out_dt))


# speedup vs baseline: 1.7514x; 1.0151x over previous
import functools

import numpy as np
import jax
import jax.numpy as jnp
from jax import lax
from jax.experimental import pallas as pl
from jax.experimental.pallas import tpu as pltpu
from jax.experimental.pallas import tpu_sc as plsc

D = 128
NC = 2
NS = 16
CH = 128


_z = np.int32(0)


def _rup(x, m):
    return (x + m - 1) // m * m


@functools.cache
def _sc_segsum(K, R, n_src):
    mesh = plsc.VectorSubcoreMesh(core_axis_name="c", subcore_axis_name="s")
    rs = R // NS

    @functools.partial(
        pl.kernel,
        mesh=mesh,
        out_type=jax.ShapeDtypeStruct((NC, R, D), jnp.float32),
        scratch_types=[
            pltpu.VMEM((K, CH), jnp.int32),
            pltpu.VMEM((K, CH), jnp.int32),
            pltpu.VMEM((CH, D), jnp.float32),
            pltpu.VMEM_SHARED((R, D), jnp.float32),
            pltpu.SemaphoreType.DMA,
        ],
    )
    def k(m_hbm, src_hbm, dst_hbm, z_hbm, out_hbm, src_v, dst_v, rows_v, acc, sem):
        cid = lax.axis_index("c")
        sid = lax.axis_index("s")
        tid = cid * NS + sid
        pltpu.sync_copy(z_hbm, acc.at[pl.ds(sid * rs, rs)])
        pltpu.sync_copy(src_hbm.at[tid], src_v)
        pltpu.sync_copy(dst_hbm.at[tid], dst_v)
        plsc.subcore_barrier()

        def body(i, carry):
            pltpu.async_copy(m_hbm.at[src_v.at[i]], rows_v, sem).wait()
            pltpu.sync_copy(rows_v, acc.at[dst_v.at[i]], add=True)
            return carry

        lax.fori_loop(jnp.int32(0), jnp.int32(K), body, jnp.int32(0))
        plsc.subcore_barrier()
        pltpu.sync_copy(acc.at[pl.ds(sid * rs, rs)],
                        out_hbm.at[cid, pl.ds(sid * rs, rs)])

    return k


@functools.cache
def _sc_segsum_pipe(K, R, n_src):
    mesh = plsc.VectorSubcoreMesh(core_axis_name="c", subcore_axis_name="s")
    rs = R // NS
    K2 = K // 2

    @functools.partial(
        pl.kernel,
        mesh=mesh,
        out_type=jax.ShapeDtypeStruct((NC, R, D), jnp.float32),
        scratch_types=[
            pltpu.VMEM((K2, CH), jnp.int32),
            pltpu.VMEM((K2, CH), jnp.int32),
            [pltpu.VMEM((CH, D), jnp.float32)] * 2,
            pltpu.VMEM_SHARED((R, D), jnp.float32),
            [pltpu.SemaphoreType.DMA] * 2,
        ],
    )
    def k(m_hbm, src_hbm, dst_hbm, z_hbm, out_hbm, src_v, dst_v, rows_v, acc, sems):
        cid = lax.axis_index("c")
        sid = lax.axis_index("s")
        tid = cid * NS + sid
        pltpu.sync_copy(z_hbm, acc.at[pl.ds(sid * rs, rs)])
        plsc.subcore_barrier()
        for half in range(2):
            h0 = np.int32(half * K2)
            pltpu.sync_copy(src_hbm.at[tid, pl.ds(h0, K2)], src_v)
            pltpu.sync_copy(dst_hbm.at[tid, pl.ds(h0, K2)], dst_v)
            for j in range(2):
                pltpu.async_copy(m_hbm.at[src_v.at[np.int32(j)]], rows_v[j],
                                 sems[j])

            def body(q, carry):
                for j in range(2):
                    i = q * np.int32(2) + np.int32(j)
                    pltpu.make_async_copy(m_hbm.at[src_v.at[i]],
                                          rows_v[j], sems[j]).wait()
                    pltpu.sync_copy(rows_v[j], acc.at[dst_v.at[i]], add=True)
                    pltpu.async_copy(m_hbm.at[src_v.at[i + np.int32(2)]],
                                     rows_v[j], sems[j])
                return carry

            lax.fori_loop(jnp.int32(0), jnp.int32(K2 // 2 - 1), body,
                          jnp.int32(0))
            for j in range(2):
                i = np.int32(K2 - 2 + j)
                pltpu.make_async_copy(m_hbm.at[src_v.at[i]],
                                      rows_v[j], sems[j]).wait()
                pltpu.sync_copy(rows_v[j], acc.at[dst_v.at[i]], add=True)
        plsc.subcore_barrier()
        pltpu.sync_copy(acc.at[pl.ds(sid * rs, rs)],
                        out_hbm.at[cid, pl.ds(sid * rs, rs)])

    return k


def _segsum(m, src, dst, n_out):
    e = src.shape[0]
    n_src = m.shape[0]
    R = _rup(n_out + 1, 8 * NS)
    ep = _rup(e, NC * NS * CH)
    K = ep // (NC * NS * CH)
    pad = ep - e
    src_p = jnp.concatenate([src, jnp.zeros((pad,), jnp.int32)]).reshape(
        NC * NS, K, CH)
    dst_p = jnp.concatenate([dst, jnp.full((pad,), n_out, jnp.int32)]).reshape(
        NC * NS, K, CH)
    zrows = jnp.zeros((R // NS, D), jnp.float32)
    spmem_need = R * D + NS * (K * CH + 2 * CH * D) + 2 ** 16
    if K % 16 == 0 and spmem_need < 2 ** 21:
        return _sc_segsum_pipe(K, R, n_src)(m, src_p, dst_p, zrows)
    return _sc_segsum(K, R, n_src)(m, src_p, dst_p, zrows)


@functools.cache
def _sc_rowgather(K, n_src):
    mesh = plsc.VectorSubcoreMesh(core_axis_name="c", subcore_axis_name="s")

    @functools.partial(
        pl.kernel,
        mesh=mesh,
        out_type=jax.ShapeDtypeStruct((NC * NS * K * CH, D), jnp.float32),
        scratch_types=[
            pltpu.VMEM((K, CH), jnp.int32),
            [pltpu.VMEM((CH, D), jnp.float32)] * K,
            [pltpu.SemaphoreType.DMA] * K,
        ],
    )
    def k(u_hbm, widx_hbm, out_hbm, widx_v, rows_v, sems):
        cid = lax.axis_index("c")
        sid = lax.axis_index("s")
        tid = cid * NS + sid
        pltpu.sync_copy(widx_hbm.at[tid], widx_v)
        for j in range(K):
            pltpu.async_copy(u_hbm.at[widx_v.at[np.int32(j)]], rows_v[j],
                             sems[j])
        for j in range(K):
            pltpu.make_async_copy(u_hbm.at[widx_v.at[np.int32(j)]],
                                  rows_v[j], sems[j]).wait()
            pltpu.sync_copy(rows_v[j],
                            out_hbm.at[pl.ds((tid * K + j) * CH, CH)])

    return k


def _rowgather(table, widx):
    n = widx.shape[0]
    npad = _rup(n, NC * NS * CH)
    K = npad // (NC * NS * CH)
    widx_p = jnp.concatenate(
        [widx, jnp.zeros((npad - n,), jnp.int32)]).reshape(NC * NS, K, CH)
    return _sc_rowgather(K, table.shape[0])(table, widx_p)


def _dot(a, b):
    return lax.dot_general(a, b, (((1,), (0,)), ((), ())),
                           precision=lax.Precision.HIGHEST,
                           preferred_element_type=jnp.float32)


def _ln(z):
    mu = jnp.mean(z, axis=-1, keepdims=True)
    var = jnp.mean((z - mu) ** 2, axis=-1, keepdims=True)
    return (z - mu) * lax.rsqrt(var + 1e-5)


def _ln_mm2_body(z_ref, wm_ref, ws_ref, m_ref, s_ref):
    h = _ln(z_ref[...])
    m_ref[...] = _dot(h, wm_ref[...])
    s_ref[...] = _dot(h, ws_ref[...])


@functools.cache
def _ln_mm2(n, bn):
    grid = n // bn
    w_spec = pl.BlockSpec((D, D), lambda i: (_z, _z))
    r_spec = pl.BlockSpec((bn, D), lambda i: (i, _z))
    return pl.pallas_call(
        _ln_mm2_body,
        grid=(grid,),
        in_specs=[r_spec, w_spec, w_spec],
        out_specs=[r_spec, r_spec],
        out_shape=[jax.ShapeDtypeStruct((n, D), jnp.float32)] * 2,
    )


def _combine1_body(s_ref, aggp_ref, wup_ref, keep_ref, h_ref, u_ref, *, n):
    hc = s_ref[...] + aggp_ref[0, :n, :] + aggp_ref[1, :n, :]
    h_ref[...] = hc
    u_ref[...] = _dot(keep_ref[...] * hc, wup_ref[...])


@functools.cache
def _combine1(n, R):
    spec = pl.BlockSpec((n, D), lambda: (_z, _z))
    return pl.pallas_call(
        functools.partial(_combine1_body, n=n),
        in_specs=[spec,
                  pl.BlockSpec((NC, R, D), lambda: (_z, _z, _z)),
                  pl.BlockSpec((D, D), lambda: (_z, _z)),
                  pl.BlockSpec((n, 1), lambda: (_z, _z))],
        out_specs=[spec, spec],
        out_shape=[jax.ShapeDtypeStruct((n, D), jnp.float32)] * 2,
    )


def _assemble0_body(s_ref, aggp_ref, inpp_ref, wup_ref, bup_ref, o_ref,
                    *, final_ln):
    hc = s_ref[...] + aggp_ref[0] + aggp_ref[1]
    z = hc + inpp_ref[0] + inpp_ref[1] + _dot(hc, wup_ref[...]) + bup_ref[...]
    o_ref[...] = _ln(z) if final_ln else z


@functools.cache
def _assemble0(n, bn, R, final_ln):
    grid = n // bn
    r_spec = pl.BlockSpec((bn, D), lambda i: (i, _z))
    p_spec = pl.BlockSpec((NC, bn, D), lambda i: (_z, i, _z))
    return pl.pallas_call(
        functools.partial(_assemble0_body, final_ln=final_ln),
        grid=(grid,),
        in_specs=[r_spec, p_spec, p_spec,
                  pl.BlockSpec((D, D), lambda i: (_z, _z)),
                  pl.BlockSpec((1, D), lambda i: (_z, _z))],
        out_specs=r_spec,
        out_shape=jax.ShapeDtypeStruct((n, D), jnp.float32),
    )


def _ln_only_body(z_ref, o_ref):
    o_ref[...] = _ln(z_ref[...])


@functools.cache
def _ln_only(n):
    spec = pl.BlockSpec((n, D), lambda: (_z, _z))
    return pl.pallas_call(
        _ln_only_body,
        in_specs=[spec],
        out_specs=spec,
        out_shape=jax.ShapeDtypeStruct((n, D), jnp.float32),
    )


def kernel(hn0, hn1, Wself, Wmsg, Wup, bup, edge_index0, edge_index1, idx1):
    n0, _ = hn0.shape
    n1, _ = hn1.shape
    L = Wself.shape[0]
    out_dt = jnp.result_type(hn0.dtype, Wself.dtype, Wup.dtype)
    src0 = edge_index0[0].astype(jnp.int32)
    dst0 = edge_index0[1].astype(jnp.int32)
    src1 = edge_index1[0].astype(jnp.int32)
    dst1 = edge_index1[1].astype(jnp.int32)
    idx1 = idx1.astype(jnp.int32)
    Wself = Wself.astype(jnp.float32)
    Wmsg = Wmsg.astype(jnp.float32)
    Wup = Wup.astype(jnp.float32)
    bup = bup.astype(jnp.float32)

    ar = jnp.arange(n1, dtype=jnp.int32)
    winner = jnp.full((n0,), -1, jnp.int32).at[idx1].max(ar,
                                                         mode='promise_in_bounds')
    keep = (winner[idx1] == ar).astype(jnp.float32)[:, None]

    bn0 = 1000
    R0 = _rup(n0 + 1, 8 * NS)
    R1 = _rup(n1 + 1, 8 * NS)

    z0, z1 = hn0.astype(jnp.float32), hn1.astype(jnp.float32)
    for l in range(L):
        m0, s0 = _ln_mm2(n0, bn0)(z0, Wmsg[l, 0], Wself[l, 0])
        m1, s1 = _ln_mm2(n1, n1)(z1, Wmsg[l, 1], Wself[l, 1])
        agg0p = _segsum(m0, src0, dst0, n0)
        agg1p = _segsum(m1, src1, dst1, n1)
        h1c, u1 = _combine1(n1, R1)(s1, agg1p, Wup[l, :D], keep)
        inpp = _segsum(u1, ar, idx1, n0)
        z0 = _assemble0(n0, bn0, R0, l == L - 1)(
            s0, agg0p, inpp, Wup[l, D:], bup[l][None, :])
        z1 = h1c
    return (z0.astype(out_dt), _ln_only(n1)(z1).astype(out_dt))
